# baseline (speedup 1.0000x reference)
import jax
import jax.numpy as jnp
from jax import lax
from jax.experimental import pallas as pl
from jax.experimental.pallas import tpu as pltpu
from jax.experimental.pallas import tpu_sc as plsc

_L = 16
_NC = 2
_NS = 16
_NW = _NC * _NS
_B = 16384
_V = 1000000
_D = 64
_PD = 2 * _D
_BPW = _B // _NW
_CHUNK = 128
_NCH = _BPW // _CHUNK
_CB = 512
_TGRID = (_V + 2 * _CB - 1) // (2 * _CB)


def _transpose_body(xa_ref, xb_ref, o_ref):
    ii = lax.broadcasted_iota(jnp.int32, (_D, _D), 0)
    jj = lax.broadcasted_iota(jnp.int32, (_D, _D), 1)
    ident = (ii == jj).astype(jnp.float32)
    dn = (((0,), (0,)), ((), ()))
    o_ref[:, 0:_D] = lax.dot_general(
        xa_ref[...], ident, dn, preferred_element_type=jnp.float32)
    o_ref[:, _D:_PD] = lax.dot_general(
        xb_ref[...], ident, dn, preferred_element_type=jnp.float32)


_tc_pack = pl.pallas_call(
    _transpose_body,
    grid=(_TGRID,),
    in_specs=[
        pl.BlockSpec((_D, _CB), lambda i: (0, 2 * i)),
        pl.BlockSpec((_D, _CB), lambda i: (0, 2 * i + 1)),
    ],
    out_specs=pl.BlockSpec((_CB, _PD), lambda i: (i, 0)),
    out_shape=jax.ShapeDtypeStruct((_TGRID * _CB, _PD), jnp.float32),
)


def _sc_body(uidx_hbm, iidx_hbm, utab_hbm, itab_hbm, out_hbm,
             uidx_v, iidx_v, updx_v, ipdx_v, ubuf, ibuf, out_v, sems):
    wid = lax.axis_index("s") * _NC + lax.axis_index("c")
    base = wid * _BPW

    pltpu.sync_copy(uidx_hbm.at[pl.ds(base, _BPW)], uidx_v)
    pltpu.sync_copy(iidx_hbm.at[pl.ds(base, _BPW)], iidx_v)

    def pack_body(g, _):
        sl = pl.ds(g * _L, _L)
        u = uidx_v[sl]
        i = iidx_v[sl]
        updx_v[sl] = jnp.bitwise_or(
            lax.shift_left(lax.shift_right_logical(u, 10), 9),
            jnp.bitwise_and(u, 511))
        ipdx_v[sl] = jnp.bitwise_or(
            lax.shift_left(lax.shift_right_logical(i, 10), 9),
            jnp.bitwise_and(i, 511))
        return _

    lax.fori_loop(0, _BPW // _L, pack_body, 0)

    def fire(c):
        s = sems.at[c % 2]
        return (
            pltpu.async_copy(utab_hbm.at[updx_v.at[pl.ds(c * _CHUNK, _CHUNK)]],
                             ubuf.at[c % 2], s),
            pltpu.async_copy(itab_hbm.at[ipdx_v.at[pl.ds(c * _CHUNK, _CHUNK)]],
                             ibuf.at[c % 2], s),
        )

    lane = lax.iota(jnp.int32, _L)
    rots = [jnp.bitwise_and(lane + (1 << t), _L - 1) for t in range(4)]

    def compute_chunk(c):
        ub = ubuf.at[c % 2]
        ib = ibuf.at[c % 2]

        def group_body(g, _):
            r0 = g * _L
            gsl = pl.ds(c * _CHUNK + r0, _L)
            huv = (lax.shift_right_logical(uidx_v[gsl], 9) & 1) * _D
            hiv = (lax.shift_right_logical(iidx_v[gsl], 9) & 1) * _D
            accv = jnp.zeros((_L,), jnp.float32)
            for j in range(_L):
                r = r0 + j
                hu = huv[j]
                hi = hiv[j]
                p = (ub[r, pl.ds(hu, _L)] * ib[r, pl.ds(hi, _L)])
                for k in range(1, _D // _L):
                    p = p + (ub[r, pl.ds(hu + k * _L, _L)]
                             * ib[r, pl.ds(hi + k * _L, _L)])
                for t in range(4):
                    p = p + jnp.take(p, rots[t], axis=0)
                accv = jnp.where(lane == j, p, accv)
            out_v[pl.ds(c * _CHUNK + r0, _L)] = accv
            return _

        lax.fori_loop(0, _CHUNK // _L, group_body, 0)

    pending = fire(0)
    for c in range(_NCH):
        nxt = fire(c + 1) if c + 1 < _NCH else ()
        for cp in pending:
            cp.wait()
        compute_chunk(c)
        pending = nxt

    pltpu.sync_copy(out_v, out_hbm.at[pl.ds(base, _BPW)])


@jax.jit
def _run(user_indices, item_indices, user_table, item_table):
    ut = user_table.T
    it = item_table.T
    utab2 = _tc_pack(ut, ut)
    itab2 = _tc_pack(it, it)
    mesh = plsc.VectorSubcoreMesh(core_axis_name="c", subcore_axis_name="s")
    f = pl.kernel(
        _sc_body,
        mesh=mesh,
        out_type=jax.ShapeDtypeStruct((_B,), jnp.float32),
        scratch_types=[
            pltpu.VMEM((_BPW,), jnp.int32),
            pltpu.VMEM((_BPW,), jnp.int32),
            pltpu.VMEM((_BPW,), jnp.int32),
            pltpu.VMEM((_BPW,), jnp.int32),
            pltpu.VMEM((2, _CHUNK, _PD), jnp.float32),
            pltpu.VMEM((2, _CHUNK, _PD), jnp.float32),
            pltpu.VMEM((_BPW,), jnp.float32),
            pltpu.SemaphoreType.DMA((2,)),
        ],
    )
    return f(user_indices, item_indices, utab2, itab2)


def kernel(user_indices, item_indices, user_table, item_table):
    return _run(user_indices.astype(jnp.int32), item_indices.astype(jnp.int32),
                user_table, item_table)


# speedup vs baseline: 3.4158x; 3.4158x over previous
import jax
import jax.numpy as jnp
from jax import lax
from jax.experimental import pallas as pl
from jax.experimental.pallas import tpu as pltpu
from jax.experimental.pallas import tpu_sc as plsc

_L = 16
_NC = 2
_NS = 16
_NW = _NC * _NS
_B = 16384
_V = 1000000
_D = 64
_BPW = _B // _NW
_NBUF = 4


def _sc_body(uidx_hbm, iidx_hbm, ut_hbm, it_hbm, out_hbm,
             uidx_v, iidx_v, ublk, iblk, out_v, usem, isem):
    wid = lax.axis_index("s") * _NC + lax.axis_index("c")
    base = wid * _BPW

    pltpu.sync_copy(uidx_hbm.at[pl.ds(base, _BPW)], uidx_v)
    pltpu.sync_copy(iidx_hbm.at[pl.ds(base, _BPW)], iidx_v)

    lane = lax.iota(jnp.int32, _L)

    def group_body(g, _):
        g16 = g * _L
        uvec = uidx_v[pl.ds(g16, _L)]
        ivec = iidx_v[pl.ds(g16, _L)]

        def fire(j):
            slot = j % _NBUF
            ru = uvec[j]
            ri = ivec[j]
            cu0 = pl.multiple_of(lax.shift_right_logical(ru, 7) * 128, 128)
            ci0 = pl.multiple_of(lax.shift_right_logical(ri, 7) * 128, 128)
            return (
                pltpu.async_copy(ut_hbm.at[:, pl.ds(cu0, 128)],
                                 ublk.at[slot], usem.at[slot]),
                pltpu.async_copy(it_hbm.at[:, pl.ds(ci0, 128)],
                                 iblk.at[slot], isem.at[slot]),
            )

        pending = [fire(j) for j in range(_NBUF)]
        accv = jnp.zeros((_L,), jnp.float32)
        for j in range(_L):
            slot = j % _NBUF
            cpu, cpi = pending[slot]
            cpu.wait()
            cpi.wait()
            cu = jnp.bitwise_and(uvec[j], 127)
            ci = jnp.bitwise_and(ivec[j], 127)
            owu = jnp.bitwise_and(cu, 127 - 15)
            owi = jnp.bitwise_and(ci, 127 - 15)
            lu = jnp.full((_L,), 0, jnp.int32) + jnp.bitwise_and(cu, 15)
            li = jnp.full((_L,), 0, jnp.int32) + jnp.bitwise_and(ci, 15)

            def dim_body(k, a):
                for dd in range(_L):
                    d = k * _L + dd
                    uwin = ublk[slot, d, pl.ds(owu, _L)]
                    iwin = iblk[slot, d, pl.ds(owi, _L)]
                    a = a + jnp.take(uwin, lu, axis=0) * jnp.take(iwin, li, axis=0)
                return a

            acc = lax.fori_loop(0, _D // _L, dim_body,
                                jnp.zeros((_L,), jnp.float32))
            accv = jnp.where(lane == j, acc, accv)
            if j + _NBUF < _L:
                pending[slot] = fire(j + _NBUF)
        out_v[pl.ds(g16, _L)] = accv
        return _

    lax.fori_loop(0, _BPW // _L, group_body, 0)

    pltpu.sync_copy(out_v, out_hbm.at[pl.ds(base, _BPW)])


@jax.jit
def _run(user_indices, item_indices, ut, it):
    mesh = plsc.VectorSubcoreMesh(core_axis_name="c", subcore_axis_name="s")
    f = pl.kernel(
        _sc_body,
        mesh=mesh,
        out_type=jax.ShapeDtypeStruct((_B,), jnp.float32),
        scratch_types=[
            pltpu.VMEM((_BPW,), jnp.int32),
            pltpu.VMEM((_BPW,), jnp.int32),
            pltpu.VMEM((_NBUF, _D, 128), jnp.float32),
            pltpu.VMEM((_NBUF, _D, 128), jnp.float32),
            pltpu.VMEM((_BPW,), jnp.float32),
            pltpu.SemaphoreType.DMA((_NBUF,)),
            pltpu.SemaphoreType.DMA((_NBUF,)),
        ],
    )
    return f(user_indices, item_indices, ut, it)


def kernel(user_indices, item_indices, user_table, item_table):
    return _run(user_indices.astype(jnp.int32), item_indices.astype(jnp.int32),
                user_table.T, item_table.T)


# speedup vs baseline: 3.4549x; 1.0115x over previous
import jax
import jax.numpy as jnp
from jax import lax
from jax.experimental import pallas as pl
from jax.experimental.pallas import tpu as pltpu
from jax.experimental.pallas import tpu_sc as plsc

_L = 16
_NC = 2
_NS = 16
_NW = _NC * _NS
_B = 16384
_V = 1000000
_D = 64
_BPW = _B // _NW
_NBUF = 6


def _sc_body(uidx_hbm, iidx_hbm, ut_hbm, it_hbm, out_hbm,
             uidx_v, iidx_v, ublk, iblk, out_v, usem, isem):
    wid = lax.axis_index("s") * _NC + lax.axis_index("c")
    base = wid * _BPW

    pltpu.sync_copy(uidx_hbm.at[pl.ds(base, _BPW)], uidx_v)
    pltpu.sync_copy(iidx_hbm.at[pl.ds(base, _BPW)], iidx_v)

    lane = lax.iota(jnp.int32, _L)

    def group_body(g, _):
        g16 = g * _L
        uvec = uidx_v[pl.ds(g16, _L)]
        ivec = iidx_v[pl.ds(g16, _L)]

        def fire(j):
            slot = j % _NBUF
            ru = uvec[j]
            ri = ivec[j]
            cu0 = pl.multiple_of(lax.shift_right_logical(ru, 7) * 128, 128)
            ci0 = pl.multiple_of(lax.shift_right_logical(ri, 7) * 128, 128)
            return (
                pltpu.async_copy(ut_hbm.at[:, pl.ds(cu0, 128)],
                                 ublk.at[slot], usem.at[slot]),
                pltpu.async_copy(it_hbm.at[:, pl.ds(ci0, 128)],
                                 iblk.at[slot], isem.at[slot]),
            )

        pending = [fire(j) for j in range(_NBUF)]
        accv = jnp.zeros((_L,), jnp.float32)
        for j in range(_L):
            slot = j % _NBUF
            cpu, cpi = pending[slot]
            cpu.wait()
            cpi.wait()
            cu = jnp.bitwise_and(uvec[j], 127)
            ci = jnp.bitwise_and(ivec[j], 127)
            owu = jnp.bitwise_and(cu, 127 - 15)
            owi = jnp.bitwise_and(ci, 127 - 15)
            lu = jnp.full((_L,), 0, jnp.int32) + jnp.bitwise_and(cu, 15)
            li = jnp.full((_L,), 0, jnp.int32) + jnp.bitwise_and(ci, 15)

            def dim_body(k, a):
                for dd in range(_L):
                    d = k * _L + dd
                    uwin = ublk[slot, d, pl.ds(owu, _L)]
                    iwin = iblk[slot, d, pl.ds(owi, _L)]
                    a = a + jnp.take(uwin, lu, axis=0) * jnp.take(iwin, li, axis=0)
                return a

            acc = lax.fori_loop(0, _D // _L, dim_body,
                                jnp.zeros((_L,), jnp.float32))
            accv = jnp.where(lane == j, acc, accv)
            if j + _NBUF < _L:
                pending[slot] = fire(j + _NBUF)
        out_v[pl.ds(g16, _L)] = accv
        return _

    lax.fori_loop(0, _BPW // _L, group_body, 0)

    pltpu.sync_copy(out_v, out_hbm.at[pl.ds(base, _BPW)])


@jax.jit
def _run(user_indices, item_indices, ut, it):
    mesh = plsc.VectorSubcoreMesh(core_axis_name="c", subcore_axis_name="s")
    f = pl.kernel(
        _sc_body,
        mesh=mesh,
        out_type=jax.ShapeDtypeStruct((_B,), jnp.float32),
        scratch_types=[
            pltpu.VMEM((_BPW,), jnp.int32),
            pltpu.VMEM((_BPW,), jnp.int32),
            pltpu.VMEM((_NBUF, _D, 128), jnp.float32),
            pltpu.VMEM((_NBUF, _D, 128), jnp.float32),
            pltpu.VMEM((_BPW,), jnp.float32),
            pltpu.SemaphoreType.DMA((_NBUF,)),
            pltpu.SemaphoreType.DMA((_NBUF,)),
        ],
    )
    return f(user_indices, item_indices, ut, it)


def kernel(user_indices, item_indices, user_table, item_table):
    return _run(user_indices.astype(jnp.int32), item_indices.astype(jnp.int32),
                user_table.T, item_table.T)


# speedup vs baseline: 3.4604x; 1.0016x over previous
import jax
import jax.numpy as jnp
from jax import lax
from jax.experimental import pallas as pl
from jax.experimental.pallas import tpu as pltpu
from jax.experimental.pallas import tpu_sc as plsc

_L = 16
_NC = 2
_NS = 16
_NW = _NC * _NS
_B = 16384
_V = 1000000
_D = 64
_BPW = _B // _NW
_NBUF = 7


def _sc_body(uidx_hbm, iidx_hbm, ut_hbm, it_hbm, out_hbm,
             uidx_v, iidx_v, ublk, iblk, out_v, usem, isem):
    wid = lax.axis_index("s") * _NC + lax.axis_index("c")
    base = wid * _BPW

    pltpu.sync_copy(uidx_hbm.at[pl.ds(base, _BPW)], uidx_v)
    pltpu.sync_copy(iidx_hbm.at[pl.ds(base, _BPW)], iidx_v)

    lane = lax.iota(jnp.int32, _L)

    def group_body(g, _):
        g16 = g * _L
        uvec = uidx_v[pl.ds(g16, _L)]
        ivec = iidx_v[pl.ds(g16, _L)]

        def fire(j):
            slot = j % _NBUF
            ru = uvec[j]
            ri = ivec[j]
            cu0 = pl.multiple_of(lax.shift_right_logical(ru, 7) * 128, 128)
            ci0 = pl.multiple_of(lax.shift_right_logical(ri, 7) * 128, 128)
            return (
                pltpu.async_copy(ut_hbm.at[:, pl.ds(cu0, 128)],
                                 ublk.at[slot], usem.at[slot]),
                pltpu.async_copy(it_hbm.at[:, pl.ds(ci0, 128)],
                                 iblk.at[slot], isem.at[slot]),
            )

        pending = [fire(j) for j in range(_NBUF)]
        accv = jnp.zeros((_L,), jnp.float32)
        for j in range(_L):
            slot = j % _NBUF
            cpu, cpi = pending[slot]
            cpu.wait()
            cpi.wait()
            cu = jnp.bitwise_and(uvec[j], 127)
            ci = jnp.bitwise_and(ivec[j], 127)
            owu = jnp.bitwise_and(cu, 127 - 15)
            owi = jnp.bitwise_and(ci, 127 - 15)
            lu = jnp.full((_L,), 0, jnp.int32) + jnp.bitwise_and(cu, 15)
            li = jnp.full((_L,), 0, jnp.int32) + jnp.bitwise_and(ci, 15)

            def dim_body(k, a):
                for dd in range(_L):
                    d = k * _L + dd
                    uwin = ublk[slot, d, pl.ds(owu, _L)]
                    iwin = iblk[slot, d, pl.ds(owi, _L)]
                    a = a + jnp.take(uwin, lu, axis=0) * jnp.take(iwin, li, axis=0)
                return a

            acc = lax.fori_loop(0, _D // _L, dim_body,
                                jnp.zeros((_L,), jnp.float32))
            accv = jnp.where(lane == j, acc, accv)
            if j + _NBUF < _L:
                pending[slot] = fire(j + _NBUF)
        out_v[pl.ds(g16, _L)] = accv
        return _

    lax.fori_loop(0, _BPW // _L, group_body, 0)

    pltpu.sync_copy(out_v, out_hbm.at[pl.ds(base, _BPW)])


@jax.jit
def _run(user_indices, item_indices, ut, it):
    mesh = plsc.VectorSubcoreMesh(core_axis_name="c", subcore_axis_name="s")
    f = pl.kernel(
        _sc_body,
        mesh=mesh,
        out_type=jax.ShapeDtypeStruct((_B,), jnp.float32),
        scratch_types=[
            pltpu.VMEM((_BPW,), jnp.int32),
            pltpu.VMEM((_BPW,), jnp.int32),
            pltpu.VMEM((_NBUF, _D, 128), jnp.float32),
            pltpu.VMEM((_NBUF, _D, 128), jnp.float32),
            pltpu.VMEM((_BPW,), jnp.float32),
            pltpu.SemaphoreType.DMA((_NBUF,)),
            pltpu.SemaphoreType.DMA((_NBUF,)),
        ],
    )
    return f(user_indices, item_indices, ut, it)


def kernel(user_indices, item_indices, user_table, item_table):
    return _run(user_indices.astype(jnp.int32), item_indices.astype(jnp.int32),
                user_table.T, item_table.T)
